# baseline (speedup 1.0000x reference)
import functools

import jax
import jax.numpy as jnp
from jax import lax
from jax.experimental import pallas as pl
from jax.experimental.pallas import tpu as pltpu
from jax.experimental.pallas import tpu_sc as plsc

N = 100000
D = 128
HIDDEN = 256
OUT = 10
S = 512

CHUNK = 128
NW = 32
SC_CHUNKS = 448
CPT = SC_CHUNKS // NW
NBUF = 7
SC_ROWS = SC_CHUNKS * CHUNK
TC_BLOCK = 512
TC_BLOCKS = -(-(N - SC_ROWS) // TC_BLOCK)
N_PAD = SC_ROWS + TC_BLOCKS * TC_BLOCK
ROWS_PER_TILE_OUT = S // 16


def _seg_sum_sc(x, ids, zeros):
    mesh = plsc.VectorSubcoreMesh(core_axis_name="c", subcore_axis_name="s")

    @functools.partial(
        pl.kernel,
        out_type=jax.ShapeDtypeStruct((2, S, D), jnp.float32),
        mesh=mesh,
        scratch_types=[
            pltpu.VMEM((NBUF, CHUNK, D), jnp.float32),
            pltpu.VMEM((CPT, 1, CHUNK), jnp.int32),
            pltpu.VMEM_SHARED((S, D), jnp.float32),
            pltpu.SemaphoreType.DMA((NBUF,)),
            pltpu.SemaphoreType.DMA,
            pltpu.SemaphoreType.DMA((NBUF,)),
            pltpu.SemaphoreType.DMA,
        ],
    )
    def seg_kernel(x_hbm, ids_hbm, zeros_hbm, out_hbm,
                   rows_v, idx_v, acc_sh, rsems, isem, ssems, sem0):
        cid = lax.axis_index("c")
        sid = lax.axis_index("s")
        wid = sid * 2 + cid
        c0 = wid * CPT

        def issue(s):
            b = s % NBUF
            pltpu.async_copy(
                x_hbm.at[pl.ds(pl.multiple_of((c0 + s) * CHUNK, CHUNK),
                               CHUNK), :],
                rows_v.at[b], rsems.at[b])

        def wait_staged(s):
            b = s % NBUF
            pltpu.make_async_copy(
                x_hbm.at[pl.ds(0, CHUNK), :], rows_v.at[b], rsems.at[b]
            ).wait()
            pltpu.make_async_copy(
                ids_hbm.at[pl.ds(0, CHUNK)], idx_v.at[s, 0], isem).wait()

        def scatter(s):
            b = s % NBUF
            pltpu.async_copy(rows_v.at[b], acc_sh.at[idx_v.at[s, 0]],
                             ssems.at[b], add=True)

        def wait_scatter(s):
            b = s % NBUF
            pltpu.make_async_copy(rows_v.at[b], acc_sh.at[idx_v.at[s, 0]],
                                  ssems.at[b]).wait()

        row0 = sid * ROWS_PER_TILE_OUT
        zdesc = pltpu.async_copy(
            zeros_hbm.at[pl.ds(row0, ROWS_PER_TILE_OUT), :],
            acc_sh.at[pl.ds(row0, ROWS_PER_TILE_OUT), :], sem0)
        for s in range(CPT):
            pltpu.async_copy(
                ids_hbm.at[pl.ds(pl.multiple_of((c0 + s) * CHUNK, CHUNK),
                                 CHUNK)],
                idx_v.at[s, 0], isem)
        for s in range(NBUF):
            issue(s)
        zdesc.wait()
        plsc.subcore_barrier()

        for s in range(CPT):
            wait_staged(s)
            scatter(s)
            prev = s - 1
            nxt = prev + NBUF
            if prev >= 0 and nxt < CPT:
                wait_scatter(prev)
                issue(nxt)

        for s in range(max(CPT - NBUF, 0), CPT):
            wait_scatter(s)

        plsc.subcore_barrier()

        pltpu.sync_copy(acc_sh.at[pl.ds(row0, ROWS_PER_TILE_OUT), :],
                        out_hbm.at[cid, pl.ds(row0, ROWS_PER_TILE_OUT), :])

    return seg_kernel(x, ids, zeros)


def _pool_tc(x, ids_pad):
    def pool_kernel(ids_ref, x_ref, o_ref):
        j = pl.program_id(0)
        rows = x_ref[...]
        gidx = SC_ROWS + j * TC_BLOCK + lax.broadcasted_iota(
            jnp.int32, (TC_BLOCK, D), 0)
        rows = jnp.where(gidx < N, rows, 0.0)
        hi = rows.astype(jnp.bfloat16)
        lo = (rows - hi.astype(jnp.float32)).astype(jnp.bfloat16)
        ids = ids_ref[...]
        seg = lax.broadcasted_iota(jnp.int32, (S, TC_BLOCK), 0)
        oneh = (ids == seg).astype(jnp.bfloat16)
        acc = (jnp.dot(oneh, hi, preferred_element_type=jnp.float32)
               + jnp.dot(oneh, lo, preferred_element_type=jnp.float32))

        @pl.when(j == 0)
        def _():
            o_ref[...] = acc

        @pl.when(j > 0)
        def _():
            o_ref[...] += acc

    base = SC_ROWS // TC_BLOCK
    return pl.pallas_call(
        pool_kernel,
        grid=(TC_BLOCKS,),
        in_specs=[
            pl.BlockSpec((1, TC_BLOCK), lambda j: (0, base + j)),
            pl.BlockSpec((TC_BLOCK, D), lambda j: (base + j, 0)),
        ],
        out_specs=pl.BlockSpec((S, D), lambda j: (0, 0)),
        out_shape=jax.ShapeDtypeStruct((S, D), jnp.float32),
    )(ids_pad, x)


def _mlp_tc(sc_partials, tc_partial, W1, b1, W2, b2):
    def mlp_kernel(p_ref, t_ref, W1_ref, b1_ref, W2_ref, b2_ref, o_ref):
        pooled = p_ref[0] + p_ref[1] + t_ref[...]
        h = jnp.dot(pooled, W1_ref[...], preferred_element_type=jnp.float32)
        h = jnp.maximum(h + b1_ref[...][None, :], 0.0)
        logits = jnp.dot(h, W2_ref[...], preferred_element_type=jnp.float32)
        logits = logits + b2_ref[...][None, :]
        m = jnp.max(logits, axis=-1, keepdims=True)
        shifted = logits - m
        lse = jnp.log(jnp.sum(jnp.exp(shifted), axis=-1, keepdims=True))
        o_ref[...] = shifted - lse

    return pl.pallas_call(
        mlp_kernel,
        out_shape=jax.ShapeDtypeStruct((S, OUT), jnp.float32),
    )(sc_partials, tc_partial, W1, b1, W2, b2)


def kernel(x, batch, W1, b1, W2, b2):
    ids = batch.astype(jnp.int32)
    ids_pad = jnp.full((1, N_PAD), 999, jnp.int32)
    ids_pad = lax.dynamic_update_slice(ids_pad, ids[None, :], (0, 0))
    zeros = jnp.zeros((S, D), dtype=jnp.float32)
    sc_partials = _seg_sum_sc(x, ids, zeros)
    tc_partial = _pool_tc(x, ids_pad)
    return _mlp_tc(sc_partials, tc_partial, W1, b1, W2, b2)


# speedup vs baseline: 1.6355x; 1.6355x over previous
import functools

import jax
import jax.numpy as jnp
from jax import lax
from jax.experimental import pallas as pl
from jax.experimental.pallas import tpu as pltpu
from jax.experimental.pallas import tpu_sc as plsc

N = 100000
D = 128
HIDDEN = 256
OUT = 10
S = 512

CHUNK = 128
NW = 32
SC_CHUNKS = 768
CPT = SC_CHUNKS // NW
NBUF = 7
SC_ROWS = SC_CHUNKS * CHUNK
TC_BLOCK = 512
TC_BLOCKS = -(-(N - SC_ROWS) // TC_BLOCK)
N_PAD = SC_ROWS + TC_BLOCKS * TC_BLOCK
ROWS_PER_TILE_OUT = S // 16


def _seg_sum_sc(x, ids, zeros):
    mesh = plsc.VectorSubcoreMesh(core_axis_name="c", subcore_axis_name="s")

    @functools.partial(
        pl.kernel,
        out_type=jax.ShapeDtypeStruct((2, S, D), jnp.float32),
        mesh=mesh,
        scratch_types=[
            pltpu.VMEM((NBUF, CHUNK, D), jnp.float32),
            pltpu.VMEM((CPT, 1, CHUNK), jnp.int32),
            pltpu.VMEM_SHARED((S, D), jnp.float32),
            pltpu.SemaphoreType.DMA((NBUF,)),
            pltpu.SemaphoreType.DMA,
            pltpu.SemaphoreType.DMA((NBUF,)),
            pltpu.SemaphoreType.DMA,
        ],
    )
    def seg_kernel(x_hbm, ids_hbm, zeros_hbm, out_hbm,
                   rows_v, idx_v, acc_sh, rsems, isem, ssems, sem0):
        cid = lax.axis_index("c")
        sid = lax.axis_index("s")
        wid = sid * 2 + cid
        c0 = wid * CPT

        def issue(s):
            b = s % NBUF
            pltpu.async_copy(
                x_hbm.at[pl.ds(pl.multiple_of((c0 + s) * CHUNK, CHUNK),
                               CHUNK), :],
                rows_v.at[b], rsems.at[b])

        def wait_staged(s):
            b = s % NBUF
            pltpu.make_async_copy(
                x_hbm.at[pl.ds(0, CHUNK), :], rows_v.at[b], rsems.at[b]
            ).wait()
            pltpu.make_async_copy(
                ids_hbm.at[pl.ds(0, CHUNK)], idx_v.at[s, 0], isem).wait()

        def scatter(s):
            b = s % NBUF
            pltpu.async_copy(rows_v.at[b], acc_sh.at[idx_v.at[s, 0]],
                             ssems.at[b], add=True)

        def wait_scatter(s):
            b = s % NBUF
            pltpu.make_async_copy(rows_v.at[b], acc_sh.at[idx_v.at[s, 0]],
                                  ssems.at[b]).wait()

        row0 = sid * ROWS_PER_TILE_OUT
        zdesc = pltpu.async_copy(
            zeros_hbm.at[pl.ds(row0, ROWS_PER_TILE_OUT), :],
            acc_sh.at[pl.ds(row0, ROWS_PER_TILE_OUT), :], sem0)
        for s in range(CPT):
            pltpu.async_copy(
                ids_hbm.at[pl.ds(pl.multiple_of((c0 + s) * CHUNK, CHUNK),
                                 CHUNK)],
                idx_v.at[s, 0], isem)
        for s in range(NBUF):
            issue(s)
        zdesc.wait()
        plsc.subcore_barrier()

        for s in range(CPT):
            wait_staged(s)
            scatter(s)
            prev = s - 1
            nxt = prev + NBUF
            if prev >= 0 and nxt < CPT:
                wait_scatter(prev)
                issue(nxt)

        for s in range(max(CPT - NBUF, 0), CPT):
            wait_scatter(s)

        plsc.subcore_barrier()

        pltpu.sync_copy(acc_sh.at[pl.ds(row0, ROWS_PER_TILE_OUT), :],
                        out_hbm.at[cid, pl.ds(row0, ROWS_PER_TILE_OUT), :])

    return seg_kernel(x, ids, zeros)


def _pool_tc(x, ids_pad):
    def pool_kernel(ids_ref, x_ref, o_ref):
        j = pl.program_id(0)
        rows = x_ref[...]
        gidx = SC_ROWS + j * TC_BLOCK + lax.broadcasted_iota(
            jnp.int32, (TC_BLOCK, D), 0)
        rows = jnp.where(gidx < N, rows, 0.0)
        hi = rows.astype(jnp.bfloat16)
        lo = (rows - hi.astype(jnp.float32)).astype(jnp.bfloat16)
        ids = ids_ref[...]
        seg = lax.broadcasted_iota(jnp.int32, (S, TC_BLOCK), 0)
        oneh = (ids == seg).astype(jnp.bfloat16)
        acc = (jnp.dot(oneh, hi, preferred_element_type=jnp.float32)
               + jnp.dot(oneh, lo, preferred_element_type=jnp.float32))

        @pl.when(j == 0)
        def _():
            o_ref[...] = acc

        @pl.when(j > 0)
        def _():
            o_ref[...] += acc

    base = SC_ROWS // TC_BLOCK
    return pl.pallas_call(
        pool_kernel,
        grid=(TC_BLOCKS,),
        in_specs=[
            pl.BlockSpec((1, TC_BLOCK), lambda j: (0, base + j)),
            pl.BlockSpec((TC_BLOCK, D), lambda j: (base + j, 0)),
        ],
        out_specs=pl.BlockSpec((S, D), lambda j: (0, 0)),
        out_shape=jax.ShapeDtypeStruct((S, D), jnp.float32),
    )(ids_pad, x)


def _mlp_tc(sc_partials, tc_partial, W1, b1, W2, b2):
    def mlp_kernel(p_ref, t_ref, W1_ref, b1_ref, W2_ref, b2_ref, o_ref):
        pooled = p_ref[0] + p_ref[1] + t_ref[...]
        h = jnp.dot(pooled, W1_ref[...], preferred_element_type=jnp.float32)
        h = jnp.maximum(h + b1_ref[...][None, :], 0.0)
        logits = jnp.dot(h, W2_ref[...], preferred_element_type=jnp.float32)
        logits = logits + b2_ref[...][None, :]
        m = jnp.max(logits, axis=-1, keepdims=True)
        shifted = logits - m
        lse = jnp.log(jnp.sum(jnp.exp(shifted), axis=-1, keepdims=True))
        o_ref[...] = shifted - lse

    return pl.pallas_call(
        mlp_kernel,
        out_shape=jax.ShapeDtypeStruct((S, OUT), jnp.float32),
    )(sc_partials, tc_partial, W1, b1, W2, b2)


def kernel(x, batch, W1, b1, W2, b2):
    ids = batch.astype(jnp.int32)
    ids_pad = jnp.full((1, N_PAD), 999, jnp.int32)
    ids_pad = lax.dynamic_update_slice(ids_pad, ids[None, :], (0, 0))
    zeros = jnp.zeros((S, D), dtype=jnp.float32)
    sc_partials = _seg_sum_sc(x, ids, zeros)
    tc_partial = _pool_tc(x, ids_pad)
    return _mlp_tc(sc_partials, tc_partial, W1, b1, W2, b2)
